# Initial kernel scaffold; baseline (speedup 1.0000x reference)
import jax
import jax.numpy as jnp
from jax.experimental import pallas as pl


def kernel(X_one_hot_2d, A_t, Y, t_float_E, batch_src, batch_dst, batch_E_one_hot, W_e, b_e, W_o, b_o, Wc1, Wc2):
    raise NotImplementedError("write your pallas kernel here")


# speedup vs baseline: 1.7114x; 1.7114x over previous
import functools

import jax
import jax.numpy as jnp
from jax import lax
from jax.experimental import pallas as pl
from jax.experimental.pallas import tpu as pltpu
from jax.experimental.pallas import tpu_sc as plsc

N = 4096
E = 65536
DX = 512
H = 1024
CE = 5
CY = 10

NC = 2
NS = 16
NW = NC * NS


def _proj_body(x_ref, w1_ref, w2_ref, o1_ref, o2_ref):
    x = x_ref[...]
    o1_ref[...] = jnp.dot(x, w1_ref[...], preferred_element_type=jnp.float32)
    o2_ref[...] = jnp.dot(x, w2_ref[...], preferred_element_type=jnp.float32)


def _project(X, W1, W2):
    BM = 512
    return pl.pallas_call(
        _proj_body,
        grid=(N // BM,),
        in_specs=[
            pl.BlockSpec((BM, DX), lambda i: (i, 0)),
            pl.BlockSpec((DX, H), lambda i: (0, 0)),
            pl.BlockSpec((DX, H), lambda i: (0, 0)),
        ],
        out_specs=[
            pl.BlockSpec((BM, H), lambda i: (i, 0)),
            pl.BlockSpec((BM, H), lambda i: (i, 0)),
        ],
        out_shape=[jax.ShapeDtypeStruct((N, H), jnp.float32)] * 2,
    )(X, W1, W2)


_CH = 32


def _sc_gather(T1, T2, src, dst):
    mesh = plsc.VectorSubcoreMesh(core_axis_name="c", subcore_axis_name="s")
    bpw = E // NW

    @functools.partial(
        pl.kernel,
        out_type=[jax.ShapeDtypeStruct((E, H), jnp.float32)] * 2,
        mesh=mesh,
        scratch_types=[
            pltpu.VMEM((_CH,), jnp.int32),
            pltpu.VMEM((_CH,), jnp.int32),
            pltpu.VMEM((_CH, H), jnp.float32),
            pltpu.VMEM((_CH, H), jnp.float32),
            pltpu.SemaphoreType.DMA,
            pltpu.SemaphoreType.DMA,
        ],
    )
    def gather_kernel(t1, t2, s_h, d_h, g1, g2, i1_v, i2_v, r1_v, r2_v,
                      sem1, sem2):
        wid = lax.axis_index("s") * NC + lax.axis_index("c")
        base = wid * bpw

        def chunk(c, carry):
            off = base + c * _CH
            pltpu.sync_copy(s_h.at[pl.ds(off, _CH)], i1_v)
            pltpu.sync_copy(d_h.at[pl.ds(off, _CH)], i2_v)
            cp1 = pltpu.async_copy(t1.at[i1_v], r1_v, sem1)
            cp2 = pltpu.async_copy(t2.at[i2_v], r2_v, sem2)
            cp1.wait()
            cp2.wait()
            pltpu.sync_copy(r1_v, g1.at[pl.ds(off, _CH)])
            pltpu.sync_copy(r2_v, g2.at[pl.ds(off, _CH)])
            return carry

        lax.fori_loop(0, bpw // _CH, chunk, 0)

    return gather_kernel(T1, T2, src, dst)


_TE = 2048


def _edge_head(G1, G2, At, tf, W5, W6, be, Wo, bo, gum, eoh):
    grid = (E // _TE,)

    def body(g1_ref, g2_ref, at_ref, t_ref, w5_ref, w6_ref, be_ref, wo_ref,
             bo_ref, gum_ref, eoh_ref, b_ref, le_ref):
        i = pl.program_id(0)
        sm = jnp.dot(at_ref[...], w5_ref[...],
                     preferred_element_type=jnp.float32)
        sm = sm + t_ref[...] * w6_ref[...]
        h = jnp.maximum(g1_ref[...] + g2_ref[...] + sm + be_ref[...], 0.0)
        logit = jnp.dot(h, wo_ref[...],
                        preferred_element_type=jnp.float32) + bo_ref[...]
        col = lax.broadcasted_iota(jnp.int32, (_TE, CE), 1)
        z = logit + gum_ref[...]
        zmax = jnp.max(z, axis=1, keepdims=True)
        samp = jnp.min(jnp.where(z >= zmax, col, CE), axis=1)
        b_ref[...] = (samp != 0).astype(jnp.float32)[None, None, :]
        eoh = eoh_ref[...]
        emax = jnp.max(eoh, axis=1, keepdims=True)
        te_idx = jnp.min(jnp.where(eoh >= emax, col, CE), axis=1)
        lmax = jnp.max(logit, axis=1, keepdims=True)
        lse = jnp.log(jnp.sum(jnp.exp(logit - lmax), axis=1)) + lmax[:, 0]
        lp_t = jnp.sum(jnp.where(col == te_idx[:, None], logit, 0.0),
                       axis=1) - lse
        part = -jnp.sum(lp_t) * (1.0 / E)

        @pl.when(i == 0)
        def _():
            le_ref[...] = jnp.zeros_like(le_ref)

        le_ref[...] += part[None, None]

    return pl.pallas_call(
        body,
        grid=grid,
        in_specs=[
            pl.BlockSpec((_TE, H), lambda i: (i, 0)),
            pl.BlockSpec((_TE, H), lambda i: (i, 0)),
            pl.BlockSpec((_TE, CE), lambda i: (i, 0)),
            pl.BlockSpec((_TE, 1), lambda i: (i, 0)),
            pl.BlockSpec((CE, H), lambda i: (0, 0)),
            pl.BlockSpec((1, H), lambda i: (0, 0)),
            pl.BlockSpec((1, H), lambda i: (0, 0)),
            pl.BlockSpec((H, CE), lambda i: (0, 0)),
            pl.BlockSpec((1, CE), lambda i: (0, 0)),
            pl.BlockSpec((_TE, CE), lambda i: (i, 0)),
            pl.BlockSpec((_TE, CE), lambda i: (i, 0)),
        ],
        out_specs=[
            pl.BlockSpec((1, 1, _TE), lambda i: (i, 0, 0)),
            pl.BlockSpec((1, 1), lambda i: (0, 0)),
        ],
        out_shape=[
            jax.ShapeDtypeStruct((E // _TE, 1, _TE), jnp.float32),
            jax.ShapeDtypeStruct((1, 1), jnp.float32),
        ],
    )(G1, G2, At, tf, W5, W6, be, Wo, bo, gum, eoh)


_ZCH = 16384
_SCB = 128


def _sc_scatter(src, dst, bvals):
    mesh = plsc.VectorSubcoreMesh(core_axis_name="c", subcore_axis_name="s")
    epw = E // NS
    n_sc = epw // _SCB

    @functools.partial(
        pl.kernel,
        out_type=jax.ShapeDtypeStruct((N * N,), jnp.float32),
        mesh=mesh,
        scratch_types=[
            pltpu.VMEM((_ZCH,), jnp.float32),
            pltpu.VMEM((epw,), jnp.int32),
            pltpu.VMEM((epw,), jnp.int32),
            pltpu.VMEM((epw,), jnp.float32),
            pltpu.VMEM((n_sc, _SCB), jnp.int32),
            pltpu.VMEM((n_sc, _SCB), jnp.int32),
            pltpu.SemaphoreType.DMA,
            pltpu.SemaphoreType.DMA,
        ],
    )
    def scatter_kernel(s_h, d_h, b_h, a_h, z_v, s_v, d_v, v_v, i1_v, i2_v,
                       sem1, sem2):
        cid = lax.axis_index("c")
        sid = lax.axis_index("s")

        @pl.when(cid == 0)
        def _zero():
            def zb(i, carry):
                z_v[pl.ds(i * 16, 16)] = jnp.zeros((16,), jnp.float32)
                return carry

            lax.fori_loop(0, _ZCH // 16, zb, 0)
            words = (N * N) // NS
            zbase = sid * words

            def zc(i, carry):
                pltpu.sync_copy(z_v, a_h.at[pl.ds(zbase + i * _ZCH, _ZCH)])
                return carry

            lax.fori_loop(0, words // _ZCH, zc, 0)

        plsc.subcore_barrier()

        @pl.when(cid == 0)
        def _scatter():
            ebase = sid * epw
            pltpu.sync_copy(s_h.at[pl.ds(ebase, epw)], s_v)
            pltpu.sync_copy(d_h.at[pl.ds(ebase, epw)], d_v)
            pltpu.sync_copy(b_h.at[pl.ds(ebase, epw)], v_v)

            def ixrow(j, carry):
                def ix(i, c2):
                    sv = s_v[pl.ds(j * _SCB + i * 16, 16)]
                    dv = d_v[pl.ds(j * _SCB + i * 16, 16)]
                    i1_v[j, pl.ds(i * 16, 16)] = dv * N + sv
                    i2_v[j, pl.ds(i * 16, 16)] = sv * N + dv
                    return c2

                lax.fori_loop(0, _SCB // 16, ix, 0)
                return carry

            lax.fori_loop(0, n_sc, ixrow, 0)

            def sc(j, carry):
                vseg = v_v.at[pl.ds(j * _SCB, _SCB)]
                cp1 = pltpu.async_copy(vseg, a_h.at[i1_v.at[j]], sem1)
                cp2 = pltpu.async_copy(vseg, a_h.at[i2_v.at[j]], sem2)
                cp1.wait()
                cp2.wait()
                return carry

            lax.fori_loop(0, n_sc, sc, 0)

    return scatter_kernel(src, dst, bvals)


_BM5 = 256


def _classifier(A, X, Wc1, Wc2, Y3):
    def body(a_ref, x_ref, w1_ref, w2_ref, y_ref, ly_ref):
        i = pl.program_id(0)
        a = a_ref[...]
        row = lax.broadcasted_iota(jnp.int32, (_BM5, N), 0) + i * _BM5
        coln = lax.broadcasted_iota(jnp.int32, (_BM5, N), 1)
        a = jnp.maximum(a, (row == coln).astype(jnp.float32))
        agg = jnp.dot(a, x_ref[...], preferred_element_type=jnp.float32)
        hy = jnp.maximum(
            jnp.dot(agg, w1_ref[...], preferred_element_type=jnp.float32),
            0.0)
        ly = jnp.dot(hy, w2_ref[...], preferred_element_type=jnp.float32)
        yb = y_ref[0, 0, :]
        lmax = jnp.max(ly, axis=1, keepdims=True)
        lse = jnp.log(jnp.sum(jnp.exp(ly - lmax), axis=1)) + lmax[:, 0]
        c10 = lax.broadcasted_iota(jnp.int32, (_BM5, CY), 1)
        lp_t = jnp.sum(jnp.where(c10 == yb[:, None], ly, 0.0), axis=1) - lse
        part = -jnp.sum(lp_t) * (1.0 / N)

        @pl.when(i == 0)
        def _():
            ly_ref[...] = jnp.zeros_like(ly_ref)

        ly_ref[...] += part[None, None]

    return pl.pallas_call(
        body,
        grid=(N // _BM5,),
        in_specs=[
            pl.BlockSpec((_BM5, N), lambda i: (i, 0)),
            pl.BlockSpec((N, DX), lambda i: (0, 0)),
            pl.BlockSpec((DX, H), lambda i: (0, 0)),
            pl.BlockSpec((H, CY), lambda i: (0, 0)),
            pl.BlockSpec((1, 1, _BM5), lambda i: (i, 0, 0)),
        ],
        out_specs=pl.BlockSpec((1, 1), lambda i: (0, 0)),
        out_shape=jax.ShapeDtypeStruct((1, 1), jnp.float32),
    )(A, X, Wc1, Wc2, Y3)


def kernel(X_one_hot_2d, A_t, Y, t_float_E, batch_src, batch_dst,
           batch_E_one_hot, W_e, b_e, W_o, b_o, Wc1, Wc2):
    src = batch_src.astype(jnp.int32)
    dst = batch_dst.astype(jnp.int32)
    W1 = W_e[:DX]
    W2 = W_e[DX:2 * DX]
    W5 = W_e[2 * DX:2 * DX + CE]
    W6 = W_e[2 * DX + CE:].reshape(1, H)
    be = b_e.reshape(1, H)
    bo = b_o.reshape(1, CE)
    gum = jax.random.gumbel(jax.random.key(42), (E, CE), jnp.float32)

    XW1, XW2 = _project(X_one_hot_2d, W1, W2)
    G1, G2 = _sc_gather(XW1, XW2, src, dst)
    bflag3, loss_e = _edge_head(G1, G2, A_t, t_float_E, W5, W6, be, W_o, bo,
                                gum, batch_E_one_hot)
    bflag = bflag3.reshape(E)
    A_flat = _sc_scatter(src, dst, bflag)
    A = A_flat.reshape(N, N)
    Y3 = Y.astype(jnp.int32).reshape(N // _BM5, 1, _BM5)
    loss_y = _classifier(A, X_one_hot_2d, Wc1, Wc2, Y3)
    return loss_e[0, 0], loss_y[0, 0]
